# Initial kernel scaffold; baseline (speedup 1.0000x reference)
import functools

import jax
import jax.numpy as jnp
from jax import lax
from jax.experimental import pallas as pl
from jax.experimental.pallas import tpu as pltpu
from jax.experimental.pallas import tpu_sc as plsc

_N = 10000
_D = 256
_H = 128
_B = 128
_NB = 79
_EPT = _B * _NB
_E_PAD = 16 * _EPT


def _halfmm_body(x_ref, w_ref, o_ref):
    o_ref[0] = lax.dot_general(
        x_ref[...], w_ref[0], (((1,), (1,)), ((), ())),
        preferred_element_type=jnp.float32,
        precision=lax.Precision.HIGHEST,
    )


def _matmul_halves(x, w):
    n, k = x.shape
    w2 = w.reshape(2, _H, k)
    bm = 1000
    return pl.pallas_call(
        _halfmm_body,
        grid=(n // bm, 2),
        in_specs=[
            pl.BlockSpec((bm, k), lambda i, g: (i, 0)),
            pl.BlockSpec((1, _H, k), lambda i, g: (g, 0, 0)),
        ],
        out_specs=pl.BlockSpec((1, bm, _H), lambda i, g: (g, i, 0)),
        out_shape=jax.ShapeDtypeStruct((2, n, _H), jnp.float32),
    )(x, w2)


def _fuse_body(x_ref, w_ref, b_ref, acc_ref, rs_ref, o_ref):
    y = lax.dot_general(
        x_ref[...], w_ref[0], (((1,), (1,)), ((), ())),
        preferred_element_type=jnp.float32,
        precision=lax.Precision.HIGHEST,
    )
    y = y + b_ref[0][None, :] + acc_ref[0]
    y = jnp.maximum(y, 0.0)
    mask = (rs_ref[...] > 0.0).astype(jnp.float32)
    o_ref[...] = y * mask


def _fuse(x, w, b, acc, rowsum):
    n, k = x.shape
    w2 = w.reshape(2, _H, k)
    b2 = b.reshape(2, _H)
    rs = rowsum.reshape(n, 1)
    bm = 1000
    return pl.pallas_call(
        _fuse_body,
        grid=(n // bm, 2),
        in_specs=[
            pl.BlockSpec((bm, k), lambda i, g: (i, 0)),
            pl.BlockSpec((1, _H, k), lambda i, g: (g, 0, 0)),
            pl.BlockSpec((1, _H), lambda i, g: (g, 0)),
            pl.BlockSpec((1, bm, _H), lambda i, g: (g, i, 0)),
            pl.BlockSpec((bm, 1), lambda i, g: (i, 0)),
        ],
        out_specs=pl.BlockSpec((bm, _H), lambda i, g: (i, g)),
        out_shape=jax.ShapeDtypeStruct((n, _D), jnp.float32),
    )(x, w2, b2, acc, rs)


def _sc_spmm(table_u, table_i, rows_u, cols_u, vals_u,
             rows_i, cols_i, vals_i, zeros2d, zeros1d):
    mesh = plsc.VectorSubcoreMesh(core_axis_name="c", subcore_axis_name="s")

    @functools.partial(
        pl.kernel,
        mesh=mesh,
        out_type=[
            jax.ShapeDtypeStruct((2, _N, _H), jnp.float32),
            jax.ShapeDtypeStruct((2, _N, _H), jnp.float32),
            jax.ShapeDtypeStruct((_N,), jnp.float32),
            jax.ShapeDtypeStruct((_N,), jnp.float32),
        ],
        scratch_types=[
            pltpu.VMEM((_B,), jnp.int32),
            pltpu.VMEM((_B,), jnp.int32),
            pltpu.VMEM((_B,), jnp.float32),
            pltpu.VMEM((_B, _H), jnp.float32),
            pltpu.VMEM_SHARED((_N, _H), jnp.float32),
            pltpu.VMEM_SHARED((_N,), jnp.float32),
            pltpu.SemaphoreType.DMA,
        ],
    )
    def k(tu_h, ti_h, ru_h, cu_h, vu_h, ri_h, ci_h, vi_h, z2_h, z1_h,
          accu_h, acci_h, rsu_h, rsi_h,
          col_v, row_v, val_v, rows_buf, acc_sp, rs_sp, sem):
        c = lax.axis_index("c")
        s = lax.axis_index("s")
        core_off = c * _N

        def run_pass(table_h, rows_h, cols_h, vals_h, acc_out_h, rs_out_h):
            pltpu.sync_copy(z2_h.at[pl.ds(s * 625, 625)],
                            acc_sp.at[pl.ds(s * 625, 625)])

            @pl.when(jnp.logical_and(c == 0, s < 10))
            def _():
                pltpu.sync_copy(z1_h.at[pl.ds(s * 1000, 1000)],
                                rs_sp.at[pl.ds(s * 1000, 1000)])

            plsc.subcore_barrier()

            base = s * _EPT

            def batch(b, carry):
                off = base + b * _B
                pltpu.sync_copy(cols_h.at[pl.ds(off, _B)], col_v)
                pltpu.sync_copy(rows_h.at[pl.ds(off, _B)], row_v)
                pltpu.sync_copy(vals_h.at[pl.ds(off, _B)], val_v)
                for kk in range(_B // 16):
                    sl = pl.ds(kk * 16, 16)
                    col_v[sl] = col_v[sl] + core_off
                pltpu.async_copy(table_h.at[col_v], rows_buf, sem).wait()

                def scale(e, carry2):
                    v = plsc.load_gather(val_v, [jnp.full((16,), e, jnp.int32)])
                    for j in range(_H // 16):
                        slj = pl.ds(j * 16, 16)
                        rows_buf[e, slj] = rows_buf[e, slj] * v
                    return carry2

                lax.fori_loop(0, _B, scale, 0)
                pltpu.sync_copy(rows_buf, acc_sp.at[row_v], add=True)

                @pl.when(c == 0)
                def _():
                    pltpu.sync_copy(val_v, rs_sp.at[row_v], add=True)

                return carry

            lax.fori_loop(0, _NB, batch, 0)
            plsc.subcore_barrier()

            pltpu.sync_copy(acc_sp.at[pl.ds(s * 625, 625)],
                            acc_out_h.at[c, pl.ds(s * 625, 625)])

            @pl.when(jnp.logical_and(c == 0, s < 10))
            def _():
                pltpu.sync_copy(rs_sp.at[pl.ds(s * 1000, 1000)],
                                rs_out_h.at[pl.ds(s * 1000, 1000)])

            plsc.subcore_barrier()

        run_pass(tu_h, ru_h, cu_h, vu_h, accu_h, rsu_h)
        run_pass(ti_h, ri_h, ci_h, vi_h, acci_h, rsi_h)

    return k(table_u, table_i, rows_u, cols_u, vals_u,
             rows_i, cols_i, vals_i, zeros2d, zeros1d)


def _pad_edges(index, values):
    rows = index[0].astype(jnp.int32)
    cols = index[1].astype(jnp.int32)
    pad = _E_PAD - rows.shape[0]
    rows = jnp.concatenate([rows, jnp.zeros((pad,), jnp.int32)])
    cols = jnp.concatenate([cols, jnp.zeros((pad,), jnp.int32)])
    vals = jnp.concatenate([values.astype(jnp.float32),
                            jnp.zeros((pad,), jnp.float32)])
    return rows, cols, vals


def kernel(xu_t, xi_t, i2u_index, i2u_values, u2i_index, u2i_values,
           W_uu, b_uu, W_ii, b_ii, W_ui, W_iu):
    iu2 = _matmul_halves(xi_t, W_iu)
    ui2 = _matmul_halves(xu_t, W_ui)
    table_u = iu2.reshape(2 * _N, _H)
    table_i = ui2.reshape(2 * _N, _H)

    ru, cu, vu = _pad_edges(i2u_index, i2u_values)
    ri, ci, vi = _pad_edges(u2i_index, u2i_values)
    z2 = jnp.zeros((_N, _H), jnp.float32)
    z1 = jnp.zeros((_N,), jnp.float32)

    acc_u, acc_i, rs_u, rs_i = _sc_spmm(
        table_u, table_i, ru, cu, vu, ri, ci, vi, z2, z1)

    delta_u = _fuse(xu_t, W_uu, b_uu, acc_u, rs_u)
    delta_i = _fuse(xi_t, W_ii, b_ii, acc_i, rs_i)
    return (delta_u, delta_i)


# speedup vs baseline: 2.3634x; 2.3634x over previous
import functools

import jax
import jax.numpy as jnp
from jax import lax
from jax.experimental import pallas as pl
from jax.experimental.pallas import tpu as pltpu
from jax.experimental.pallas import tpu_sc as plsc

_N = 10000
_D = 256
_H = 128
_B = 128
_NB = 79
_EPT = _B * _NB
_E_PAD = 16 * _EPT


def _halfmm_body(x_ref, w_ref, o_ref):
    o_ref[0] = lax.dot_general(
        x_ref[...], w_ref[0], (((1,), (1,)), ((), ())),
        preferred_element_type=jnp.float32,
        precision=lax.Precision.HIGHEST,
    )


def _matmul_halves(x, w):
    n, k = x.shape
    w2 = w.reshape(2, _H, k)
    bm = 1000
    return pl.pallas_call(
        _halfmm_body,
        grid=(n // bm, 2),
        in_specs=[
            pl.BlockSpec((bm, k), lambda i, g: (i, 0)),
            pl.BlockSpec((1, _H, k), lambda i, g: (g, 0, 0)),
        ],
        out_specs=pl.BlockSpec((1, bm, _H), lambda i, g: (g, i, 0)),
        out_shape=jax.ShapeDtypeStruct((2, n, _H), jnp.float32),
    )(x, w2)


def _fuse_body(x_ref, w_ref, b_ref, acc_ref, rs_ref, o_ref):
    y = lax.dot_general(
        x_ref[...], w_ref[0], (((1,), (1,)), ((), ())),
        preferred_element_type=jnp.float32,
        precision=lax.Precision.HIGHEST,
    )
    y = y + b_ref[0] + acc_ref[0]
    y = jnp.maximum(y, 0.0)
    mask = (rs_ref[...] > 0.0).astype(jnp.float32)
    o_ref[...] = y * mask


def _fuse(x, w, b, acc, rowsum):
    n, k = x.shape
    w2 = w.reshape(2, _H, k)
    b2 = b.reshape(2, 1, _H)
    rs = rowsum.reshape(n, 1)
    bm = 1000
    return pl.pallas_call(
        _fuse_body,
        grid=(n // bm, 2),
        in_specs=[
            pl.BlockSpec((bm, k), lambda i, g: (i, 0)),
            pl.BlockSpec((1, _H, k), lambda i, g: (g, 0, 0)),
            pl.BlockSpec((1, 1, _H), lambda i, g: (g, 0, 0)),
            pl.BlockSpec((1, bm, _H), lambda i, g: (g, i, 0)),
            pl.BlockSpec((bm, 1), lambda i, g: (i, 0)),
        ],
        out_specs=pl.BlockSpec((bm, _H), lambda i, g: (i, g)),
        out_shape=jax.ShapeDtypeStruct((n, _D), jnp.float32),
    )(x, w2, b2, acc, rs)


def _sc_spmm(table_u, table_i, rows_u, cols_u, vals_u,
             rows_i, cols_i, vals_i, zeros2d, zeros1d):
    mesh = plsc.VectorSubcoreMesh(core_axis_name="c", subcore_axis_name="s")

    @functools.partial(
        pl.kernel,
        mesh=mesh,
        out_type=[
            jax.ShapeDtypeStruct((2, _N, _H), jnp.float32),
            jax.ShapeDtypeStruct((2, _N, _H), jnp.float32),
            jax.ShapeDtypeStruct((_N,), jnp.float32),
            jax.ShapeDtypeStruct((_N,), jnp.float32),
        ],
        scratch_types=[
            pltpu.VMEM((_B,), jnp.int32),
            pltpu.VMEM((_B,), jnp.int32),
            pltpu.VMEM((_B,), jnp.float32),
            pltpu.VMEM((_B, _H), jnp.float32),
            pltpu.VMEM_SHARED((_N, _H), jnp.float32),
            pltpu.VMEM_SHARED((_N,), jnp.float32),
            pltpu.VMEM((1000,), jnp.float32),
            pltpu.SemaphoreType.DMA,
        ],
    )
    def k(tu_h, ti_h, ru_h, cu_h, vu_h, ri_h, ci_h, vi_h, z2_h, z1_h,
          accu_h, acci_h, rsu_h, rsi_h,
          col_v, row_v, val_v, rows_buf, acc_sp, rs_sp, rs_stage, sem):
        c = lax.axis_index("c")
        s = lax.axis_index("s")
        core_off = c * _N

        def run_pass(table_h, rows_h, cols_h, vals_h, acc_out_h, rs_out_h):
            @pl.when(s < 15)
            def _():
                pltpu.sync_copy(z2_h.at[pl.ds(s * 632, 632)],
                                acc_sp.at[pl.ds(s * 632, 632)])

            @pl.when(s == 15)
            def _():
                pltpu.sync_copy(z2_h.at[pl.ds(9480, 520)],
                                acc_sp.at[pl.ds(9480, 520)])

            @pl.when(jnp.logical_and(c == 0, s < 10))
            def _():
                pltpu.sync_copy(z1_h.at[pl.ds(s * 1000, 1000)], rs_stage)
                pltpu.sync_copy(rs_stage, rs_sp.at[pl.ds(s * 1000, 1000)])

            plsc.subcore_barrier()

            base = s * _EPT

            def batch(b, carry):
                off = base + b * _B
                pltpu.sync_copy(cols_h.at[pl.ds(off, _B)], col_v)
                pltpu.sync_copy(rows_h.at[pl.ds(off, _B)], row_v)
                pltpu.sync_copy(vals_h.at[pl.ds(off, _B)], val_v)
                for kk in range(_B // 16):
                    sl = pl.ds(kk * 16, 16)
                    col_v[sl] = col_v[sl] + core_off
                pltpu.async_copy(table_h.at[col_v], rows_buf, sem).wait()

                def scale(g, carry2):
                    v16 = val_v[pl.ds(g * 16, 16)]
                    for l in range(16):
                        e = g * 16 + l
                        v = v16[l]
                        for j in range(_H // 16):
                            slj = pl.ds(j * 16, 16)
                            rows_buf[e, slj] = rows_buf[e, slj] * v
                    return carry2

                lax.fori_loop(0, _B // 16, scale, 0)
                pltpu.sync_copy(rows_buf, acc_sp.at[row_v], add=True)

                @pl.when(c == 0)
                def _():
                    pltpu.sync_copy(val_v, rs_sp.at[row_v], add=True)

                return carry

            lax.fori_loop(0, _NB, batch, 0)
            plsc.subcore_barrier()

            @pl.when(s < 15)
            def _():
                pltpu.sync_copy(acc_sp.at[pl.ds(s * 632, 632)],
                                acc_out_h.at[c, pl.ds(s * 632, 632)])

            @pl.when(s == 15)
            def _():
                pltpu.sync_copy(acc_sp.at[pl.ds(9480, 520)],
                                acc_out_h.at[c, pl.ds(9480, 520)])

            @pl.when(jnp.logical_and(c == 0, s < 10))
            def _():
                pltpu.sync_copy(rs_sp.at[pl.ds(s * 1000, 1000)], rs_stage)
                pltpu.sync_copy(rs_stage, rs_out_h.at[pl.ds(s * 1000, 1000)])

            plsc.subcore_barrier()

        run_pass(tu_h, ru_h, cu_h, vu_h, accu_h, rsu_h)
        run_pass(ti_h, ri_h, ci_h, vi_h, acci_h, rsi_h)

    return k(table_u, table_i, rows_u, cols_u, vals_u,
             rows_i, cols_i, vals_i, zeros2d, zeros1d)


def _pad_edges(index, values):
    rows = index[0].astype(jnp.int32)
    cols = index[1].astype(jnp.int32)
    pad = _E_PAD - rows.shape[0]
    rows = jnp.concatenate([rows, jnp.zeros((pad,), jnp.int32)])
    cols = jnp.concatenate([cols, jnp.zeros((pad,), jnp.int32)])
    vals = jnp.concatenate([values.astype(jnp.float32),
                            jnp.zeros((pad,), jnp.float32)])
    return rows, cols, vals


def kernel(xu_t, xi_t, i2u_index, i2u_values, u2i_index, u2i_values,
           W_uu, b_uu, W_ii, b_ii, W_ui, W_iu):
    iu2 = _matmul_halves(xi_t, W_iu)
    ui2 = _matmul_halves(xu_t, W_ui)
    table_u = iu2.reshape(2 * _N, _H)
    table_i = ui2.reshape(2 * _N, _H)

    ru, cu, vu = _pad_edges(i2u_index, i2u_values)
    ri, ci, vi = _pad_edges(u2i_index, u2i_values)
    z2 = jnp.zeros((_N, _H), jnp.float32)
    z1 = jnp.zeros((_N,), jnp.float32)

    acc_u, acc_i, rs_u, rs_i = _sc_spmm(
        table_u, table_i, ru, cu, vu, ri, ci, vi, z2, z1)

    delta_u = _fuse(xu_t, W_uu, b_uu, acc_u, rs_u)
    delta_i = _fuse(xi_t, W_ii, b_ii, acc_i, rs_i)
    return (delta_u, delta_i)
